# baseline (speedup 1.0000x reference)
import functools
import math

import jax
import jax.numpy as jnp
from jax import lax
from jax.experimental import pallas as pl
from jax.experimental.pallas import tpu as pltpu
from jax.experimental.pallas import tpu_sc as plsc

_LANES = 16
_NBUF = 4
_NBUF1 = 3
_CH = 128


def _pack_table(lut_t, lut_tail, V, D, NC, NS):
    NW = NC * NS
    BV = 2 * D
    n_full = V // BV
    tail = V - n_full * BV
    base_cnt = n_full // NW
    extra = n_full - base_cnt * NW
    mesh = plsc.VectorSubcoreMesh(core_axis_name="c", subcore_axis_name="s")

    @functools.partial(
        pl.kernel,
        mesh=mesh,
        compiler_params=pltpu.CompilerParams(
            use_tc_tiling_on_sc=True, needs_layout_passes=False),
        out_type=jax.ShapeDtypeStruct((V // 2, 2 * D), jnp.float32),
        scratch_types=[
            [pltpu.VMEM((D, BV), jnp.float32)] * _NBUF1,
            [pltpu.VMEM((D, BV), jnp.float32)] * _NBUF1,
            pltpu.VMEM((tail, D), jnp.float32),
            [pltpu.SemaphoreType.DMA] * _NBUF1,
            [pltpu.SemaphoreType.DMA] * _NBUF1,
        ],
    )
    def run(src_hbm, tail_hbm, out_hbm, ibufs, obufs, tbuf, gsems,
            ssems):
        wid = lax.axis_index("s") * NC + lax.axis_index("c")
        cnt = base_cnt + jnp.where(wid < extra, 1, 0).astype(jnp.int32)
        start = wid * base_cnt + lax.min(wid, jnp.int32(extra))
        iota = jax.lax.iota(jnp.int32, _LANES)
        n_iter = base_cnt + (1 if extra else 0)

        def blk_of(t):
            return jnp.where(t < cnt, start + t, start).astype(jnp.int32)

        def fire_load(slot, blk):
            pltpu.async_copy(
                src_hbm.at[:, pl.ds(pl.multiple_of(blk * BV, BV), BV)],
                ibufs[slot], gsems[slot])

        for s in range(_NBUF1):
            fire_load(s, blk_of(jnp.int32(s)))

        def transform(ibuf, obuf):
            @plsc.parallel_loop(0, _LANES, 1, unroll=8, carry=jnp.int32(0))
            def _(t, cc, ibuf=ibuf, obuf=obuf):
                rot = lax.bitwise_and(iota + t, _LANES - 1)
                for u0 in range(0, BV, _LANES):
                    uvec = rot + u0
                    rowk = lax.shift_right_logical(uvec, 1)
                    parc = lax.bitwise_and(uvec, 1) * D
                    for d0 in range(0, D, _LANES):
                        rows_d = iota + d0
                        val = plsc.load_gather(ibuf, [rows_d, uvec])
                        plsc.store_scatter(obuf, [rowk, parc + rows_d], val)
                return cc

        n_loop = -(-n_iter // _NBUF1)

        def step(it, carry):
            for s in range(_NBUF1):
                t = it * _NBUF1 + s
                blk = blk_of(t)
                pltpu.make_async_copy(
                    src_hbm.at[:, pl.ds(0, BV)], ibufs[s], gsems[s]).wait()

                @pl.when(it > 0)
                def _():
                    pltpu.make_async_copy(
                        obufs[s], out_hbm.at[pl.ds(0, D)], ssems[s]).wait()

                transform(ibufs[s], obufs[s])
                pltpu.async_copy(
                    obufs[s],
                    out_hbm.at[pl.ds(pl.multiple_of(blk * D, D), D)],
                    ssems[s])
                p = t + _NBUF1

                @pl.when(p < n_loop * _NBUF1)
                def _():
                    fire_load(s, blk_of(p))

            return carry

        lax.fori_loop(0, n_loop, step, 0)
        for s in range(_NBUF1):
            pltpu.make_async_copy(obufs[s], out_hbm.at[pl.ds(0, D)],
                                  ssems[s]).wait()

        @pl.when(wid == NW - 1)
        def _():
            pltpu.sync_copy(tail_hbm, tbuf)
            for u in range(tail):
                for j in range(D // _LANES):
                    obufs[0][u // 2,
                             pl.ds((u % 2) * D + j * _LANES, _LANES)] = (
                        tbuf[u, pl.ds(j * _LANES, _LANES)])
            pltpu.sync_copy(
                obufs[0].at[pl.ds(0, tail // 2)],
                out_hbm.at[pl.ds((V - tail) // 2, tail // 2)])

    return run(lut_t, lut_tail)


def kernel(x, lut):
    N, S = x.shape
    V, D = lut.shape
    info = plsc.get_sparse_core_info()
    NC, NS = info.num_cores, info.num_subcores
    NW = NC * NS
    per_w = N // NW
    n_blk = per_w // _CH
    n_chunks = S * n_blk
    assert N % (NW * _CH) == 0 and D % _LANES == 0
    scale = math.sqrt(D)

    xt = x.T.astype(jnp.int32)
    n_tail = V % (2 * D)
    lut2 = _pack_table(lut.T, lut[V - n_tail:], V, D, NC, NS)
    mesh = plsc.VectorSubcoreMesh(core_axis_name="c", subcore_axis_name="s")

    @functools.partial(
        pl.kernel,
        mesh=mesh,
        compiler_params=pltpu.CompilerParams(
            use_tc_tiling_on_sc=True, needs_layout_passes=False),
        out_type=jax.ShapeDtypeStruct((S, D, N), jnp.float32),
        scratch_types=[
            pltpu.VMEM((S, per_w), jnp.int32),
            [pltpu.VMEM((_CH,), jnp.int32)] * _NBUF,
            [pltpu.VMEM((_CH,), jnp.int32)] * _NBUF,
            [pltpu.VMEM((_CH, 2 * D), jnp.float32)] * _NBUF,
            [pltpu.VMEM((D, _CH), jnp.float32)] * _NBUF,
            [pltpu.SemaphoreType.DMA] * _NBUF,
            [pltpu.SemaphoreType.DMA] * _NBUF,
        ],
    )
    def run(xt_hbm, lut_hbm, out_hbm, idx_v, vhs, prs, gbufs, sbufs,
            gsems, ssems):
        wid = lax.axis_index("s") * NC + lax.axis_index("c")
        i0 = wid * per_w
        n_loop2 = -(-n_chunks // _NBUF)
        pltpu.sync_copy(xt_hbm.at[:, pl.ds(i0, per_w)], idx_v)

        iota = jax.lax.iota(jnp.int32, _LANES)

        def prep(slot, c):
            j = lax.div(c, n_blk)
            b = lax.rem(c, n_blk)
            off = b * _CH
            for k in range(_CH // _LANES):
                v = idx_v[j, pl.ds(off + k * _LANES, _LANES)]
                vhs[slot][pl.ds(k * _LANES, _LANES)] = (
                    lax.shift_right_logical(v, 1))
                prs[slot][pl.ds(k * _LANES, _LANES)] = (
                    lax.bitwise_and(v, 1) * D)

        def fire_gather(slot):
            pltpu.async_copy(lut_hbm.at[vhs[slot]], gbufs[slot], gsems[slot])

        for s in range(_NBUF):
            prep(s, jnp.int32(s))
            fire_gather(s)

        def step(it, carry):
            for s in range(_NBUF):
                c = lax.min(it * _NBUF + s, jnp.int32(n_chunks - 1))
                j = lax.div(c, n_blk)
                b = lax.rem(c, n_blk)
                gbuf, sbuf = gbufs[s], sbufs[s]
                pltpu.make_async_copy(lut_hbm.at[vhs[s]], gbuf,
                                      gsems[s]).wait()

                @pl.when(it > 0)
                def _():
                    pltpu.make_async_copy(
                        sbuf, out_hbm.at[0, :, pl.ds(0, _CH)],
                        ssems[s]).wait()

                for r0 in range(0, _CH, _LANES):
                    rows = iota + r0
                    parv = prs[s][pl.ds(r0, _LANES)]

                    @plsc.parallel_loop(0, _LANES, 1, unroll=4,
                                        carry=jnp.int32(0))
                    def _(t, cc, gbuf=gbuf, sbuf=sbuf, rows=rows,
                          parv=parv):
                        rot = lax.bitwise_and(iota + t, _LANES - 1)
                        base = parv + rot
                        for db in range(D // _LANES):
                            val = plsc.load_gather(
                                gbuf, [rows, base + db * _LANES])
                            plsc.store_scatter(
                                sbuf, [rot + db * _LANES, rows], val * scale)
                        return cc

                pltpu.async_copy(
                    sbuf, out_hbm.at[j, :, pl.ds(i0 + b * _CH, _CH)],
                    ssems[s])
                p = it * _NBUF + s + _NBUF

                @pl.when(p < n_loop2 * _NBUF)
                def _():
                    prep(s, lax.min(p, jnp.int32(n_chunks - 1)))
                    fire_gather(s)

            return carry

        lax.fori_loop(0, n_loop2, step, 0)
        for s in range(_NBUF):
            pltpu.make_async_copy(sbufs[s], out_hbm.at[0, :, pl.ds(0, _CH)],
                                  ssems[s]).wait()

    out_t = run(xt, lut2)
    return jnp.transpose(out_t, (2, 0, 1))


# speedup vs baseline: 1.0087x; 1.0087x over previous
import functools
import math

import jax
import jax.numpy as jnp
from jax import lax
from jax.experimental import pallas as pl
from jax.experimental.pallas import tpu as pltpu
from jax.experimental.pallas import tpu_sc as plsc

_LANES = 16
_NBUF = 3
_NBUF1 = 3
_CH = 128


def _pack_table(lut_t, lut_tail, V, D, NC, NS):
    NW = NC * NS
    BV = 2 * D
    n_full = V // BV
    tail = V - n_full * BV
    base_cnt = n_full // NW
    extra = n_full - base_cnt * NW
    mesh = plsc.VectorSubcoreMesh(core_axis_name="c", subcore_axis_name="s")

    @functools.partial(
        pl.kernel,
        mesh=mesh,
        compiler_params=pltpu.CompilerParams(
            use_tc_tiling_on_sc=True, needs_layout_passes=False),
        out_type=jax.ShapeDtypeStruct((V // 2, 2 * D), jnp.float32),
        scratch_types=[
            [pltpu.VMEM((D, BV), jnp.float32)] * _NBUF1,
            [pltpu.VMEM((D, BV), jnp.float32)] * _NBUF1,
            pltpu.VMEM((tail, D), jnp.float32),
            [pltpu.SemaphoreType.DMA] * _NBUF1,
            [pltpu.SemaphoreType.DMA] * _NBUF1,
        ],
    )
    def run(src_hbm, tail_hbm, out_hbm, ibufs, obufs, tbuf, gsems,
            ssems):
        wid = lax.axis_index("s") * NC + lax.axis_index("c")
        cnt = base_cnt + jnp.where(wid < extra, 1, 0).astype(jnp.int32)
        start = wid * base_cnt + lax.min(wid, jnp.int32(extra))
        iota = jax.lax.iota(jnp.int32, _LANES)
        n_iter = base_cnt + (1 if extra else 0)

        def blk_of(t):
            return jnp.where(t < cnt, start + t, start).astype(jnp.int32)

        def fire_load(slot, blk):
            pltpu.async_copy(
                src_hbm.at[:, pl.ds(pl.multiple_of(blk * BV, BV), BV)],
                ibufs[slot], gsems[slot])

        for s in range(_NBUF1):
            fire_load(s, blk_of(jnp.int32(s)))

        def transform(ibuf, obuf):
            @plsc.parallel_loop(0, _LANES, 1, unroll=8, carry=jnp.int32(0))
            def _(t, cc, ibuf=ibuf, obuf=obuf):
                rot = lax.bitwise_and(iota + t, _LANES - 1)
                for u0 in range(0, BV, _LANES):
                    uvec = rot + u0
                    rowk = lax.shift_right_logical(uvec, 1)
                    parc = lax.bitwise_and(uvec, 1) * D
                    for d0 in range(0, D, _LANES):
                        rows_d = iota + d0
                        val = plsc.load_gather(ibuf, [rows_d, uvec])
                        plsc.store_scatter(obuf, [rowk, parc + rows_d], val)
                return cc

        n_loop = -(-n_iter // _NBUF1)

        def step(it, carry):
            for s in range(_NBUF1):
                t = it * _NBUF1 + s
                blk = blk_of(t)
                pltpu.make_async_copy(
                    src_hbm.at[:, pl.ds(0, BV)], ibufs[s], gsems[s]).wait()

                @pl.when(it > 0)
                def _():
                    pltpu.make_async_copy(
                        obufs[s], out_hbm.at[pl.ds(0, D)], ssems[s]).wait()

                transform(ibufs[s], obufs[s])
                pltpu.async_copy(
                    obufs[s],
                    out_hbm.at[pl.ds(pl.multiple_of(blk * D, D), D)],
                    ssems[s])
                p = t + _NBUF1

                @pl.when(p < n_loop * _NBUF1)
                def _():
                    fire_load(s, blk_of(p))

            return carry

        lax.fori_loop(0, n_loop, step, 0)
        for s in range(_NBUF1):
            pltpu.make_async_copy(obufs[s], out_hbm.at[pl.ds(0, D)],
                                  ssems[s]).wait()

        @pl.when(wid == NW - 1)
        def _():
            pltpu.sync_copy(tail_hbm, tbuf)
            for u in range(tail):
                for j in range(D // _LANES):
                    obufs[0][u // 2,
                             pl.ds((u % 2) * D + j * _LANES, _LANES)] = (
                        tbuf[u, pl.ds(j * _LANES, _LANES)])
            pltpu.sync_copy(
                obufs[0].at[pl.ds(0, tail // 2)],
                out_hbm.at[pl.ds((V - tail) // 2, tail // 2)])

    return run(lut_t, lut_tail)


def kernel(x, lut):
    N, S = x.shape
    V, D = lut.shape
    info = plsc.get_sparse_core_info()
    NC, NS = info.num_cores, info.num_subcores
    NW = NC * NS
    per_w = N // NW
    n_blk = per_w // _CH
    n_chunks = S * n_blk
    assert N % (NW * _CH) == 0 and D % _LANES == 0
    scale = math.sqrt(D)

    xt = x.T.astype(jnp.int32)
    n_tail = V % (2 * D)
    lut2 = _pack_table(lut.T, lut[V - n_tail:], V, D, NC, NS)
    mesh = plsc.VectorSubcoreMesh(core_axis_name="c", subcore_axis_name="s")

    @functools.partial(
        pl.kernel,
        mesh=mesh,
        compiler_params=pltpu.CompilerParams(
            use_tc_tiling_on_sc=True, needs_layout_passes=False),
        out_type=jax.ShapeDtypeStruct((S, D, N), jnp.float32),
        scratch_types=[
            pltpu.VMEM((S, per_w), jnp.int32),
            [pltpu.VMEM((_CH,), jnp.int32)] * _NBUF,
            [pltpu.VMEM((_CH,), jnp.int32)] * _NBUF,
            [pltpu.VMEM((_CH, 2 * D), jnp.float32)] * _NBUF,
            [pltpu.VMEM((D, _CH), jnp.float32)] * _NBUF,
            [pltpu.SemaphoreType.DMA] * _NBUF,
            [pltpu.SemaphoreType.DMA] * _NBUF,
        ],
    )
    def run(xt_hbm, lut_hbm, out_hbm, idx_v, vhs, prs, gbufs, sbufs,
            gsems, ssems):
        wid = lax.axis_index("s") * NC + lax.axis_index("c")
        i0 = wid * per_w
        n_loop2 = -(-n_chunks // _NBUF)
        pltpu.sync_copy(xt_hbm.at[:, pl.ds(i0, per_w)], idx_v)

        iota = jax.lax.iota(jnp.int32, _LANES)

        def prep(slot, c):
            j = lax.div(c, n_blk)
            b = lax.rem(c, n_blk)
            off = b * _CH
            for k in range(_CH // _LANES):
                v = idx_v[j, pl.ds(off + k * _LANES, _LANES)]
                vhs[slot][pl.ds(k * _LANES, _LANES)] = (
                    lax.shift_right_logical(v, 1))
                prs[slot][pl.ds(k * _LANES, _LANES)] = (
                    lax.bitwise_and(v, 1) * D)

        def fire_gather(slot):
            pltpu.async_copy(lut_hbm.at[vhs[slot]], gbufs[slot], gsems[slot])

        for s in range(_NBUF):
            prep(s, jnp.int32(s))
            fire_gather(s)

        def step(it, carry):
            for s in range(_NBUF):
                c = lax.min(it * _NBUF + s, jnp.int32(n_chunks - 1))
                j = lax.div(c, n_blk)
                b = lax.rem(c, n_blk)
                gbuf, sbuf = gbufs[s], sbufs[s]
                pltpu.make_async_copy(lut_hbm.at[vhs[s]], gbuf,
                                      gsems[s]).wait()

                @pl.when(it > 0)
                def _():
                    pltpu.make_async_copy(
                        sbuf, out_hbm.at[0, :, pl.ds(0, _CH)],
                        ssems[s]).wait()

                for r0 in range(0, _CH, _LANES):
                    rows = iota + r0
                    parv = prs[s][pl.ds(r0, _LANES)]

                    @plsc.parallel_loop(0, _LANES, 1, unroll=4,
                                        carry=jnp.int32(0))
                    def _(t, cc, gbuf=gbuf, sbuf=sbuf, rows=rows,
                          parv=parv):
                        rot = lax.bitwise_and(iota + t, _LANES - 1)
                        base = parv + rot
                        for db in range(D // _LANES):
                            val = plsc.load_gather(
                                gbuf, [rows, base + db * _LANES])
                            plsc.store_scatter(
                                sbuf, [rot + db * _LANES, rows], val * scale)
                        return cc

                pltpu.async_copy(
                    sbuf, out_hbm.at[j, :, pl.ds(i0 + b * _CH, _CH)],
                    ssems[s])
                p = it * _NBUF + s + _NBUF

                @pl.when(p < n_loop2 * _NBUF)
                def _():
                    prep(s, lax.min(p, jnp.int32(n_chunks - 1)))
                    fire_gather(s)

            return carry

        lax.fori_loop(0, n_loop2, step, 0)
        for s in range(_NBUF):
            pltpu.make_async_copy(sbufs[s], out_hbm.at[0, :, pl.ds(0, _CH)],
                                  ssems[s]).wait()

    out_t = run(xt, lut2)
    return jnp.transpose(out_t, (2, 0, 1))
